# baseline (speedup 1.0000x reference)
import functools
import jax
import jax.numpy as jnp
from jax import lax
from jax.experimental import pallas as pl
from jax.experimental.pallas import tpu as pltpu
from jax.experimental.pallas import tpu_sc as plsc

_B, _T, _D = 4, 8192, 1024
_N = _B * _T
_NC, _NS = 2, 16
_NW = _NC * _NS
_B_PER_W = _N // _NW
_CHUNK = 8
_NCHUNK = _B_PER_W // _CHUNK
_NBUF = 4


@functools.partial(
    pl.kernel,
    mesh=plsc.VectorSubcoreMesh(core_axis_name="c", subcore_axis_name="s"),
    out_type=jax.ShapeDtypeStruct((_N, _D), jnp.float32),
    scratch_types=[
        pltpu.VMEM((_B_PER_W,), jnp.int32),
        pltpu.VMEM((_NBUF, _CHUNK, _D), jnp.float32),
        pltpu.VMEM_SHARED((_NS, _NBUF, _CHUNK, _D), jnp.float32),
    ] + [pltpu.SemaphoreType.DMA] * (3 * _NBUF),
)
def _gather_rows(pos_hbm, pe_hbm, out_hbm, idx_v, rows_v, rows_sh, *sems):
    cid = lax.axis_index("c")
    sid = lax.axis_index("s")
    wid = sid * _NC + cid
    base = wid * _B_PER_W
    pltpu.sync_copy(pos_hbm.at[pl.ds(base, _B_PER_W)], idx_v)

    gsems = sems[:_NBUF]
    lsems = sems[_NBUF:2 * _NBUF]
    osems = sems[2 * _NBUF:]

    def start_gather(j, b):
        off = pl.multiple_of(j * _CHUNK, _CHUNK)
        pltpu.async_copy(
            pe_hbm.at[idx_v.at[pl.ds(off, _CHUNK)]],
            rows_v.at[b],
            gsems[b],
        )

    def wait_gather(b):
        pltpu.make_async_copy(pe_hbm.at[idx_v.at[pl.ds(0, _CHUNK)]],
                              rows_v.at[b], gsems[b]).wait()

    def start_local(b):
        pltpu.async_copy(rows_v.at[b], rows_sh.at[sid, b], lsems[b])

    def wait_local(b):
        pltpu.make_async_copy(rows_v.at[b], rows_sh.at[sid, b],
                              lsems[b]).wait()

    def start_out(j, b):
        off = pl.multiple_of(base + j * _CHUNK, _CHUNK)
        pltpu.async_copy(
            rows_sh.at[sid, b],
            out_hbm.at[pl.ds(off, _CHUNK)],
            osems[b],
        )

    def wait_out(b):
        pltpu.make_async_copy(rows_sh.at[sid, b],
                              out_hbm.at[pl.ds(0, _CHUNK)], osems[b]).wait()

    def slot(j, b, c, do_pre, do_fire, do_wait_out):
        if do_pre:
            wait_local(c)
            start_out(j - 2, c)
            if do_fire:
                start_gather(j + 2, c)
        wait_gather(b)
        if do_wait_out:
            wait_out(b)
        start_local(b)

    for b in range(_NBUF):
        start_gather(b, b)
    for j in range(4):
        slot(j, j % _NBUF, (j + 2) % _NBUF, j >= 2, True, False)

    n_steady = _NCHUNK - 4 - 4
    assert n_steady % _NBUF == 0

    def body(g, _):
        j0 = 4 + g * _NBUF
        for k in range(_NBUF):
            slot(j0 + k, k, (k + 2) % _NBUF, True, True, True)
        return ()

    lax.fori_loop(0, n_steady // _NBUF, body, (), unroll=False)

    for j in range(_NCHUNK - 4, _NCHUNK):
        slot(j, j % _NBUF, (j + 2) % _NBUF, True, j + 2 < _NCHUNK, True)

    for j in (_NCHUNK, _NCHUNK + 1):
        c = (j + 2) % _NBUF
        wait_local(c)
        start_out(j - 2, c)
    for j in range(_NCHUNK - 4, _NCHUNK):
        wait_out(j % _NBUF)


def kernel(x, pe, positions):
    flat_pos = positions.reshape(_N)
    out = _gather_rows(flat_pos, pe)
    return out.reshape(_B, _T, _D).astype(x.dtype)
